# baseline (speedup 1.0000x reference)
import functools

import jax
import jax.numpy as jnp
from jax import lax
from jax.experimental import pallas as pl
from jax.experimental.pallas import tpu as pltpu
import jax.experimental.pallas.tpu_sc as plsc

_NC = 2
_NS = 16
_NW = _NC * _NS
_CT = 32
_BT = 256


def _routing_metadata(expert_indices, num_experts, bt):
    e = expert_indices.reshape(-1).astype(jnp.int32)
    p = e.shape[0]
    onehot = (e[:, None] == jnp.arange(num_experts, dtype=jnp.int32)[None, :])
    onehot_i = onehot.astype(jnp.int32)
    csum = jnp.cumsum(onehot_i, axis=0)
    counts = csum[-1]
    rank = jnp.sum(csum * onehot_i, axis=1) - 1
    padded = ((counts + bt - 1) // bt) * bt
    starts = jnp.concatenate(
        [jnp.zeros((1,), jnp.int32), jnp.cumsum(padded)[:-1].astype(jnp.int32)])
    slot = jnp.sum(onehot_i * starts[None, :], axis=1) + rank
    nb = (p + num_experts * bt) // bt
    s_used = jnp.sum(padded).astype(jnp.int32)
    block_pos = jnp.arange(nb, dtype=jnp.int32) * bt
    block_expert = jnp.clip(
        jnp.searchsorted(starts, block_pos, side="right").astype(jnp.int32) - 1,
        0, num_experts - 1)
    be_last = jnp.clip(
        jnp.searchsorted(starts, s_used - 1, side="right").astype(jnp.int32) - 1,
        0, num_experts - 1)
    valid = (block_pos < s_used).astype(jnp.int32)
    be_eff = jnp.where(valid == 1, block_expert, be_last)
    bes = jnp.stack([be_eff, valid])
    return slot, bes


def _dispatch_body(nch, tw, ct, x_hbm, pos0_hbm, pos1_hbm,
                   xrows_hbm, xbuf0, xbuf1, i00, i10, i01, i11,
                   sem00, sem10, sem01, sem11):
    wid = lax.axis_index("s") * _NC + lax.axis_index("c")
    xbufs = (xbuf0, xbuf1)
    ibufs = ((i00, i10), (i01, i11))
    sems = ((sem00, sem10), (sem01, sem11))
    waiters = [None, None]
    for c in range(nch):
        par = c % 2
        tok = wid * tw + c * ct
        if waiters[par] is not None:
            waiters[par][0].wait()
            waiters[par][1].wait()
        pltpu.sync_copy(pos0_hbm.at[wid, pl.ds(c * ct, ct)], ibufs[par][0])
        pltpu.sync_copy(pos1_hbm.at[wid, pl.ds(c * ct, ct)], ibufs[par][1])
        pltpu.sync_copy(x_hbm.at[pl.ds(tok, ct)], xbufs[par])
        cp0 = pltpu.async_copy(xbufs[par], xrows_hbm.at[ibufs[par][0]],
                               sems[par][0])
        cp1 = pltpu.async_copy(xbufs[par], xrows_hbm.at[ibufs[par][1]],
                               sems[par][1])
        waiters[par] = (cp0, cp1)
    for w in waiters:
        if w is not None:
            w[0].wait()
            w[1].wait()


def _combine_body(nch, tw, ct, hidden, y_hbm, pos0_hbm, pos1_hbm,
                  w0_hbm, w1_hbm, out_hbm,
                  b00, b10, b01, b11, w0b, w1b, i0all, i1all,
                  g00, g10, g01, g11, st0, st1):
    wid = lax.axis_index("s") * _NC + lax.axis_index("c")
    gbufs = ((b00, b10), (b01, b11))
    gsems = ((g00, g10), (g01, g11))
    ssems = (st0, st1)
    pltpu.sync_copy(pos0_hbm.at[wid], i0all)
    pltpu.sync_copy(pos1_hbm.at[wid], i1all)
    pltpu.sync_copy(w0_hbm.at[wid], w0b)
    pltpu.sync_copy(w1_hbm.at[wid], w1b)

    def issue(c):
        par = c % 2
        gd0 = pltpu.async_copy(y_hbm.at[i0all.at[pl.ds(c * ct, ct)]],
                               gbufs[par][0], gsems[par][0])
        gd1 = pltpu.async_copy(y_hbm.at[i1all.at[pl.ds(c * ct, ct)]],
                               gbufs[par][1], gsems[par][1])
        return gd0, gd1

    gwait = [None, None]
    swait = [None, None]
    gwait[0] = issue(0)
    for c in range(nch):
        par = c % 2
        if c + 1 < nch:
            par1 = (c + 1) % 2
            if swait[par1] is not None:
                swait[par1].wait()
                swait[par1] = None
            gwait[par1] = issue(c + 1)
        gwait[par][0].wait()
        gwait[par][1].wait()

        b0, b1 = gbufs[par]

        def row_combine(r, carry):
            w0 = w0b[c * ct + r, :]
            w1 = w1b[c * ct + r, :]
            for v in range(hidden // 16):
                sl = pl.ds(v * 16, 16)
                b0[r, sl] = b0[r, sl] * w0 + b1[r, sl] * w1
            return carry

        lax.fori_loop(0, ct, row_combine, 0, unroll=2)
        tok = wid * tw + c * ct
        swait[par] = pltpu.async_copy(
            b0, out_hbm.at[pl.ds(tok, ct)], ssems[par])
    for sw in swait:
        if sw is not None:
            sw.wait()


def _expert_is_new(bes_ref):
    i = pl.program_id(0)
    prev = bes_ref[0, jnp.maximum(i - 1, 0)]
    return jnp.logical_or(i == 0, bes_ref[0, i] != prev)


def _up_body(bes_ref, x_ref, u_ref, h_ref, u_bf):
    @pl.when(_expert_is_new(bes_ref))
    def _():
        u_bf[...] = u_ref[0].astype(jnp.bfloat16)

    @pl.when(bes_ref[1, pl.program_id(0)] == 1)
    def _():
        h_ref[...] = jnp.dot(
            x_ref[...].astype(jnp.bfloat16), u_bf[...],
            preferred_element_type=jnp.float32).astype(jnp.bfloat16)


def _down_body(bes_ref, h_ref, d_ref, y_ref, d_bf):
    @pl.when(_expert_is_new(bes_ref))
    def _():
        d_bf[...] = d_ref[0].astype(jnp.bfloat16)

    @pl.when(bes_ref[1, pl.program_id(0)] == 1)
    def _():
        y_ref[...] = jnp.dot(
            h_ref[...], d_bf[...], preferred_element_type=jnp.float32)


def _grouped_mlp(block_expert, x_rows, expert_up, expert_down, bt):
    s, hidden = x_rows.shape
    num_experts, _, f = expert_up.shape
    nb = s // bt
    up_spec = pltpu.PrefetchScalarGridSpec(
        num_scalar_prefetch=1,
        grid=(nb,),
        in_specs=[
            pl.BlockSpec((bt, hidden), lambda i, bes: (i, 0)),
            pl.BlockSpec((1, hidden, f), lambda i, bes: (bes[0, i], 0, 0)),
        ],
        out_specs=pl.BlockSpec((bt, f), lambda i, bes: (i, 0)),
        scratch_shapes=[pltpu.VMEM((hidden, f), jnp.bfloat16)],
    )
    h = pl.pallas_call(
        _up_body,
        grid_spec=up_spec,
        out_shape=jax.ShapeDtypeStruct((s, f), jnp.bfloat16),
    )(block_expert, x_rows, expert_up)
    down_spec = pltpu.PrefetchScalarGridSpec(
        num_scalar_prefetch=1,
        grid=(nb,),
        in_specs=[
            pl.BlockSpec((bt, f), lambda i, bes: (i, 0)),
            pl.BlockSpec((1, f, hidden), lambda i, bes: (bes[0, i], 0, 0)),
        ],
        out_specs=pl.BlockSpec((bt, hidden), lambda i, bes: (i, 0)),
        scratch_shapes=[pltpu.VMEM((f, hidden), jnp.bfloat16)],
    )
    return pl.pallas_call(
        _down_body,
        grid_spec=down_spec,
        out_shape=jax.ShapeDtypeStruct((s, hidden), jnp.float32),
    )(block_expert, h, expert_down)


def kernel(x, expert_indices, expert_weights, expert_up, expert_down):
    batch, seq, hidden = x.shape
    top_k = expert_indices.shape[-1]
    num_experts = expert_up.shape[0]
    t = batch * seq
    p = t * top_k
    s = p + num_experts * _BT
    tw = t // _NW
    nch = tw // _CT

    x_flat = x.reshape(t, hidden)
    slot, bes = _routing_metadata(expert_indices, num_experts, _BT)
    slot_tk = slot.reshape(t, top_k)
    pos0 = slot_tk[:, 0].reshape(_NW, tw)
    pos1 = slot_tk[:, 1].reshape(_NW, tw)
    w_tk = expert_weights.reshape(t, top_k).astype(jnp.float32)
    w0 = jnp.broadcast_to(w_tk[:, 0][:, None], (t, 16)).reshape(_NW, tw, 16)
    w1 = jnp.broadcast_to(w_tk[:, 1][:, None], (t, 16)).reshape(_NW, tw, 16)

    mesh = plsc.VectorSubcoreMesh(
        core_axis_name="c", subcore_axis_name="s",
        num_cores=_NC, num_subcores=_NS)

    ct_d = _CT
    nch_d = tw // ct_d
    dispatch = pl.kernel(
        functools.partial(_dispatch_body, nch_d, tw, ct_d),
        out_type=jax.ShapeDtypeStruct((s, hidden), jnp.float32),
        mesh=mesh,
        scratch_types=[
            pltpu.VMEM((ct_d, hidden), jnp.float32),
            pltpu.VMEM((ct_d, hidden), jnp.float32),
            pltpu.VMEM((ct_d,), jnp.int32),
            pltpu.VMEM((ct_d,), jnp.int32),
            pltpu.VMEM((ct_d,), jnp.int32),
            pltpu.VMEM((ct_d,), jnp.int32),
            pltpu.SemaphoreType.DMA,
            pltpu.SemaphoreType.DMA,
            pltpu.SemaphoreType.DMA,
            pltpu.SemaphoreType.DMA,
        ],
    )
    x_rows = dispatch(x_flat, pos0, pos1)

    y_rows = _grouped_mlp(bes, x_rows, expert_up, expert_down, _BT)

    ct_c = 16
    nch_c = tw // ct_c
    combine = pl.kernel(
        functools.partial(_combine_body, nch_c, tw, ct_c, hidden),
        out_type=jax.ShapeDtypeStruct((t, hidden), jnp.float32),
        mesh=mesh,
        scratch_types=[
            pltpu.VMEM((ct_c, hidden), jnp.float32),
            pltpu.VMEM((ct_c, hidden), jnp.float32),
            pltpu.VMEM((ct_c, hidden), jnp.float32),
            pltpu.VMEM((ct_c, hidden), jnp.float32),
            pltpu.VMEM((tw, 16), jnp.float32),
            pltpu.VMEM((tw, 16), jnp.float32),
            pltpu.VMEM((tw,), jnp.int32),
            pltpu.VMEM((tw,), jnp.int32),
            pltpu.SemaphoreType.DMA,
            pltpu.SemaphoreType.DMA,
            pltpu.SemaphoreType.DMA,
            pltpu.SemaphoreType.DMA,
            pltpu.SemaphoreType.DMA,
            pltpu.SemaphoreType.DMA,
        ],
    )
    out_flat = combine(y_rows, pos0, pos1, w0, w1)
    return out_flat.reshape(batch, seq, hidden)


# speedup vs baseline: 1.3955x; 1.3955x over previous
import functools

import jax
import jax.numpy as jnp
from jax import lax
from jax.experimental import pallas as pl
from jax.experimental.pallas import tpu as pltpu
import jax.experimental.pallas.tpu_sc as plsc

_NC = 2
_NS = 16
_NW = _NC * _NS
_CT = 32
_BT = 256


def _routing_metadata(expert_indices, num_experts, bt):
    e = expert_indices.reshape(-1).astype(jnp.int32)
    p = e.shape[0]
    onehot = (e[:, None] == jnp.arange(num_experts, dtype=jnp.int32)[None, :])
    onehot_i = onehot.astype(jnp.int32)
    csum = jnp.cumsum(onehot_i, axis=0)
    counts = csum[-1]
    rank = jnp.sum(csum * onehot_i, axis=1) - 1
    padded = ((counts + bt - 1) // bt) * bt
    starts = jnp.concatenate(
        [jnp.zeros((1,), jnp.int32), jnp.cumsum(padded)[:-1].astype(jnp.int32)])
    slot = jnp.sum(onehot_i * starts[None, :], axis=1) + rank
    nb = (p + num_experts * bt) // bt
    s_used = jnp.sum(padded).astype(jnp.int32)
    block_pos = jnp.arange(nb, dtype=jnp.int32) * bt
    block_expert = jnp.clip(
        jnp.searchsorted(starts, block_pos, side="right").astype(jnp.int32) - 1,
        0, num_experts - 1)
    be_last = jnp.clip(
        jnp.searchsorted(starts, s_used - 1, side="right").astype(jnp.int32) - 1,
        0, num_experts - 1)
    valid = (block_pos < s_used).astype(jnp.int32)
    be_eff = jnp.where(valid == 1, block_expert, be_last)

    bi = jnp.arange(nb, dtype=jnp.int32)
    change = jnp.concatenate(
        [jnp.zeros((1,), bool), be_eff[1:] != be_eff[:-1]])
    is_new = jnp.concatenate(
        [jnp.ones((1,), bool), change[1:]]).astype(jnp.int32)
    ordinal = jnp.cumsum(is_new) - 1
    parity = ordinal % 2
    candpos = jnp.where(change, bi, nb)
    suffix_min = jnp.flip(jax.lax.associative_scan(
        jnp.minimum, jnp.flip(candpos)))
    next_change = jnp.concatenate(
        [suffix_min[1:], jnp.full((1,), nb, jnp.int32)])
    be_pad = jnp.concatenate([be_eff, jnp.full((1,), -1, jnp.int32)])
    fetch_e = jnp.where((is_new == 1) & (next_change < nb),
                        be_pad[next_change], -1)
    fetch_p = parity ^ 1
    ctrl = jnp.stack([valid, parity, is_new, fetch_e, fetch_p, be_eff])
    return slot, ctrl


def _dispatch_body(nch, tw, ct, x_hbm, pos0_hbm, pos1_hbm,
                   xrows_hbm, xbuf0, xbuf1, i00, i10, i01, i11,
                   sem00, sem10, sem01, sem11):
    wid = lax.axis_index("s") * _NC + lax.axis_index("c")
    xbufs = (xbuf0, xbuf1)
    ibufs = ((i00, i10), (i01, i11))
    sems = ((sem00, sem10), (sem01, sem11))
    waiters = [None, None]
    for c in range(nch):
        par = c % 2
        tok = wid * tw + c * ct
        if waiters[par] is not None:
            waiters[par][0].wait()
            waiters[par][1].wait()
        pltpu.sync_copy(pos0_hbm.at[wid, pl.ds(c * ct, ct)], ibufs[par][0])
        pltpu.sync_copy(pos1_hbm.at[wid, pl.ds(c * ct, ct)], ibufs[par][1])
        pltpu.sync_copy(x_hbm.at[pl.ds(tok, ct)], xbufs[par])
        cp0 = pltpu.async_copy(xbufs[par], xrows_hbm.at[ibufs[par][0]],
                               sems[par][0])
        cp1 = pltpu.async_copy(xbufs[par], xrows_hbm.at[ibufs[par][1]],
                               sems[par][1])
        waiters[par] = (cp0, cp1)
    for w in waiters:
        if w is not None:
            w[0].wait()
            w[1].wait()


def _combine_body(nch, tw, ct, hidden, y_hbm, pos0_hbm, pos1_hbm,
                  w0_hbm, w1_hbm, out_hbm,
                  b00, b10, b01, b11, w0b, w1b, i0all, i1all,
                  g00, g10, g01, g11, st0, st1):
    wid = lax.axis_index("s") * _NC + lax.axis_index("c")
    gbufs = ((b00, b10), (b01, b11))
    gsems = ((g00, g10), (g01, g11))
    ssems = (st0, st1)
    pltpu.sync_copy(pos0_hbm.at[wid], i0all)
    pltpu.sync_copy(pos1_hbm.at[wid], i1all)
    pltpu.sync_copy(w0_hbm.at[wid], w0b)
    pltpu.sync_copy(w1_hbm.at[wid], w1b)

    def issue(c):
        par = c % 2
        gd0 = pltpu.async_copy(y_hbm.at[i0all.at[pl.ds(c * ct, ct)]],
                               gbufs[par][0], gsems[par][0])
        gd1 = pltpu.async_copy(y_hbm.at[i1all.at[pl.ds(c * ct, ct)]],
                               gbufs[par][1], gsems[par][1])
        return gd0, gd1

    gwait = [None, None]
    swait = [None, None]
    gwait[0] = issue(0)
    for c in range(nch):
        par = c % 2
        if c + 1 < nch:
            par1 = (c + 1) % 2
            if swait[par1] is not None:
                swait[par1].wait()
                swait[par1] = None
            gwait[par1] = issue(c + 1)
        gwait[par][0].wait()
        gwait[par][1].wait()

        b0, b1 = gbufs[par]

        def row_combine(r, carry):
            w0 = w0b[c * ct + r, :]
            w1 = w1b[c * ct + r, :]
            for v in range(hidden // 16):
                sl = pl.ds(v * 16, 16)
                b0[r, sl] = b0[r, sl] * w0 + b1[r, sl] * w1
            return carry

        lax.fori_loop(0, ct, row_combine, 0, unroll=2)
        tok = wid * tw + c * ct
        swait[par] = pltpu.async_copy(
            b0, out_hbm.at[pl.ds(tok, ct)], ssems[par])
    for sw in swait:
        if sw is not None:
            sw.wait()


def _mlp_body(ctrl_ref, x_ref, u_any, d_any, y_ref, u_st, d_st, su, sd):
    i = pl.program_id(0)
    par = ctrl_ref[1, i]
    fe = ctrl_ref[3, i]
    fp = ctrl_ref[4, i]

    @pl.when(i == 0)
    def _():
        cur = ctrl_ref[5, 0]
        pltpu.make_async_copy(u_any.at[pl.ds(cur, 1)],
                              u_st.at[pl.ds(0, 1)], su.at[0]).start()
        pltpu.make_async_copy(d_any.at[pl.ds(cur, 1)],
                              d_st.at[pl.ds(0, 1)], sd.at[0]).start()

    @pl.when(fe >= 0)
    def _():
        pltpu.make_async_copy(u_any.at[pl.ds(fe, 1)],
                              u_st.at[pl.ds(fp, 1)], su.at[fp]).start()
        pltpu.make_async_copy(d_any.at[pl.ds(fe, 1)],
                              d_st.at[pl.ds(fp, 1)], sd.at[fp]).start()

    @pl.when(ctrl_ref[2, i] == 1)
    def _():
        pltpu.make_async_copy(u_any.at[pl.ds(0, 1)],
                              u_st.at[pl.ds(par, 1)], su.at[par]).wait()
        pltpu.make_async_copy(d_any.at[pl.ds(0, 1)],
                              d_st.at[pl.ds(par, 1)], sd.at[par]).wait()

    @pl.when(ctrl_ref[0, i] == 1)
    def _():
        ub = u_st[pl.ds(par, 1)][0].astype(jnp.bfloat16)
        db = d_st[pl.ds(par, 1)][0].astype(jnp.bfloat16)
        h = jnp.dot(x_ref[...].astype(jnp.bfloat16), ub,
                    preferred_element_type=jnp.float32)
        y_ref[...] = jnp.dot(h.astype(jnp.bfloat16), db,
                             preferred_element_type=jnp.float32)


def _grouped_mlp(ctrl, x_rows, expert_up, expert_down, bt, interpret=False):
    s, hidden = x_rows.shape
    num_experts, _, f = expert_up.shape
    nb = s // bt
    grid_spec = pltpu.PrefetchScalarGridSpec(
        num_scalar_prefetch=1,
        grid=(nb,),
        in_specs=[
            pl.BlockSpec((bt, hidden), lambda i, c: (i, 0)),
            pl.BlockSpec(memory_space=pl.ANY),
            pl.BlockSpec(memory_space=pl.ANY),
        ],
        out_specs=pl.BlockSpec((bt, hidden), lambda i, c: (i, 0)),
        scratch_shapes=[
            pltpu.VMEM((2, hidden, f), jnp.float32),
            pltpu.VMEM((2, f, hidden), jnp.float32),
            pltpu.SemaphoreType.DMA((2,)),
            pltpu.SemaphoreType.DMA((2,)),
        ],
    )
    return pl.pallas_call(
        _mlp_body,
        grid_spec=grid_spec,
        out_shape=jax.ShapeDtypeStruct((s, hidden), jnp.float32),
        interpret=interpret,
    )(ctrl, x_rows, expert_up, expert_down)


def kernel(x, expert_indices, expert_weights, expert_up, expert_down):
    batch, seq, hidden = x.shape
    top_k = expert_indices.shape[-1]
    num_experts = expert_up.shape[0]
    t = batch * seq
    p = t * top_k
    s = p + num_experts * _BT
    tw = t // _NW
    nch = tw // _CT

    x_flat = x.reshape(t, hidden)
    slot, bes = _routing_metadata(expert_indices, num_experts, _BT)
    slot_tk = slot.reshape(t, top_k)
    pos0 = slot_tk[:, 0].reshape(_NW, tw)
    pos1 = slot_tk[:, 1].reshape(_NW, tw)
    w_tk = expert_weights.reshape(t, top_k).astype(jnp.float32)
    w0 = jnp.broadcast_to(w_tk[:, 0][:, None], (t, 16)).reshape(_NW, tw, 16)
    w1 = jnp.broadcast_to(w_tk[:, 1][:, None], (t, 16)).reshape(_NW, tw, 16)

    mesh = plsc.VectorSubcoreMesh(
        core_axis_name="c", subcore_axis_name="s",
        num_cores=_NC, num_subcores=_NS)

    ct_d = _CT
    nch_d = tw // ct_d
    dispatch = pl.kernel(
        functools.partial(_dispatch_body, nch_d, tw, ct_d),
        out_type=jax.ShapeDtypeStruct((s, hidden), jnp.float32),
        mesh=mesh,
        scratch_types=[
            pltpu.VMEM((ct_d, hidden), jnp.float32),
            pltpu.VMEM((ct_d, hidden), jnp.float32),
            pltpu.VMEM((ct_d,), jnp.int32),
            pltpu.VMEM((ct_d,), jnp.int32),
            pltpu.VMEM((ct_d,), jnp.int32),
            pltpu.VMEM((ct_d,), jnp.int32),
            pltpu.SemaphoreType.DMA,
            pltpu.SemaphoreType.DMA,
            pltpu.SemaphoreType.DMA,
            pltpu.SemaphoreType.DMA,
        ],
    )
    x_rows = dispatch(x_flat, pos0, pos1)

    y_rows = _grouped_mlp(bes, x_rows, expert_up, expert_down, _BT)

    ct_c = 16
    nch_c = tw // ct_c
    combine = pl.kernel(
        functools.partial(_combine_body, nch_c, tw, ct_c, hidden),
        out_type=jax.ShapeDtypeStruct((t, hidden), jnp.float32),
        mesh=mesh,
        scratch_types=[
            pltpu.VMEM((ct_c, hidden), jnp.float32),
            pltpu.VMEM((ct_c, hidden), jnp.float32),
            pltpu.VMEM((ct_c, hidden), jnp.float32),
            pltpu.VMEM((ct_c, hidden), jnp.float32),
            pltpu.VMEM((tw, 16), jnp.float32),
            pltpu.VMEM((tw, 16), jnp.float32),
            pltpu.VMEM((tw,), jnp.int32),
            pltpu.VMEM((tw,), jnp.int32),
            pltpu.SemaphoreType.DMA,
            pltpu.SemaphoreType.DMA,
            pltpu.SemaphoreType.DMA,
            pltpu.SemaphoreType.DMA,
            pltpu.SemaphoreType.DMA,
            pltpu.SemaphoreType.DMA,
        ],
    )
    out_flat = combine(y_rows, pos0, pos1, w0, w1)
    return out_flat.reshape(batch, seq, hidden)
